# baseline (speedup 1.0000x reference)
import jax
import jax.numpy as jnp
from jax import lax
from jax.experimental import pallas as pl
from jax.experimental.pallas import tpu as pltpu

B = 16384
C = 1000
SMOOTH = 0.1
EPS = 1e-12

CH_ROWS = 512
NCH = B // CH_ROWS
NBUF = 8
HALF = CH_ROWS // 2


def _chunk_copy(x_hbm, bufs, sems, chunk, slot):
    return pltpu.make_async_copy(
        x_hbm.at[pl.ds(chunk * CH_ROWS, CH_ROWS), :],
        bufs.at[slot],
        sems.at[slot],
    )


def _body(l_ref, x_hbm, o_ref, bufs, sems, acc_ref):
    step = pl.program_id(0)
    slot = lax.rem(step, NBUF)

    @pl.when(step == 0)
    def _():
        acc_ref[0, 0] = 0.0
        for j in range(NBUF):
            _chunk_copy(x_hbm, bufs, sems, j, j).start()

    _chunk_copy(x_hbm, bufs, sems, step, slot).wait()
    x = bufs[slot]

    nxt = step + NBUF

    @pl.when(nxt < NCH)
    def _():
        _chunk_copy(x_hbm, bufs, sems, nxt, slot).start()

    cols = lax.broadcasted_iota(jnp.int32, (CH_ROWS, C), 1)
    m = cols == l_ref[...]
    g_row = jnp.sum(jnp.where(m, x, 0.0), axis=1, keepdims=True)
    g = jnp.clip(g_row, EPS, 1.0 - EPS)
    s = (1.0 - SMOOTH) * jnp.sum(jnp.log(g) - jnp.log(1.0 - g))
    pa = jnp.clip(x[:HALF], EPS, 1.0 - EPS)
    pb = jnp.clip(x[HALF:], EPS, 1.0 - EPS)
    s += SMOOTH * jnp.sum(jnp.log(pa * pb))
    s += (1.0 - SMOOTH) * jnp.sum(jnp.log((1.0 - pa) * (1.0 - pb)))
    acc_ref[0, 0] += s

    @pl.when(step == NCH - 1)
    def _():
        o_ref[0, 0] = -acc_ref[0, 0] * (1.0 / (B * C))


def kernel(inputs, outputs, labels):
    del inputs
    lab2d = labels.astype(jnp.int32).reshape(B, 1)
    loss = pl.pallas_call(
        _body,
        grid=(NCH,),
        in_specs=[
            pl.BlockSpec((CH_ROWS, 1), lambda i: (i, 0)),
            pl.BlockSpec(memory_space=pl.ANY),
        ],
        out_specs=pl.BlockSpec((1, 1), lambda i: (0, 0),
                               memory_space=pltpu.SMEM),
        out_shape=jax.ShapeDtypeStruct((1, 1), jnp.float32),
        scratch_shapes=[
            pltpu.VMEM((NBUF, CH_ROWS, C), jnp.float32),
            pltpu.SemaphoreType.DMA((NBUF,)),
            pltpu.SMEM((1, 1), jnp.float32),
        ],
    )(lab2d, outputs)
    return loss[0, 0]
